# baseline (speedup 1.0000x reference)
import functools

import jax
import jax.numpy as jnp
from jax import lax
from jax.experimental import pallas as pl
from jax.experimental.pallas import tpu as pltpu
from jax.experimental.pallas import tpu_sc as plsc

M = 100000
C = 100
D = 128
B = 16384

_info = plsc.get_sparse_core_info()
_NC = _info.num_cores
_NS = _info.num_subcores
_NW = _NC * _NS
_BPW = B // _NW
_FCHUNK = 32
_NFCHUNK = _BPW // _FCHUNK
_NUNIT = 2 * C
_MAXJ = (_NUNIT + _NW - 1) // _NW
_OCHUNK = 2048
_NOCHUNK = B // _OCHUNK

_mesh = plsc.VectorSubcoreMesh(core_axis_name="c", subcore_axis_name="s")


@functools.partial(
    pl.kernel,
    mesh=_mesh,
    compiler_params=pltpu.CompilerParams(needs_layout_passes=False),
    out_type=[
        jax.ShapeDtypeStruct((C, B), jnp.float32),
        jax.ShapeDtypeStruct((C, B), jnp.float32),
        jax.ShapeDtypeStruct((B, D), jnp.float32),
        jax.ShapeDtypeStruct((B, D), jnp.float32),
    ],
    scratch_types=[
        pltpu.VMEM((B,), jnp.int32),
        pltpu.VMEM((M,), jnp.float32),
        [pltpu.VMEM((_OCHUNK,), jnp.float32) for _ in range(2)],
        [pltpu.VMEM((_FCHUNK,), jnp.int32) for _ in range(_NFCHUNK)],
        [pltpu.VMEM((_FCHUNK, D), jnp.float32) for _ in range(2)],
        pltpu.SemaphoreType.DMA,
        [pltpu.SemaphoreType.DMA for _ in range(2)],
        [pltpu.SemaphoreType.DMA for _ in range(2)],
        [pltpu.SemaphoreType.DMA for _ in range(2)],
    ],
)
def _gather4(wlT, slT, wf_hbm, sf_hbm, idx_hbm,
             wlT_out, slT_out, wf_out, sf_out,
             idxbuf, stage, ochunk, fidx, fbuf,
             ssem, osem, fgsem, fosem):
    wid = lax.axis_index("s") * _NC + lax.axis_index("c")
    base = wid * _BPW
    fbanks = (wf_hbm, sf_hbm)
    fouts = (wf_out, sf_out)

    idx_h = pltpu.async_copy(idx_hbm, idxbuf, ssem)
    fidx_h = [
        pltpu.async_copy(idx_hbm.at[pl.ds(base + ch * _FCHUNK, _FCHUNK)],
                         fidx[ch], fgsem[0])
        for ch in range(_NFCHUNK)
    ]
    idx_h.wait()
    for h in fidx_h:
        h.wait()

    state = {"fw": {}}

    def fire_stage(j):
        u = wid + _NW * j

        @pl.when(u < C)
        def _():
            pltpu.async_copy(wlT.at[u], stage, ssem)

        @pl.when(jnp.logical_and(u >= C, u < _NUNIT))
        def _():
            pltpu.async_copy(slT.at[u - C], stage, ssem)

    def wait_stage():
        pltpu.make_async_copy(wlT.at[0], stage, ssem).wait()

    def service_feature(ch):
        for b in range(2):
            h = state["fw"].pop(b, None)
            if h is not None:
                h.wait()
        gs = [pltpu.async_copy(fbanks[b].at[fidx[ch]], fbuf[b], fgsem[b])
              for b in range(2)]
        for b in range(2):
            gs[b].wait()
            state["fw"][b] = pltpu.async_copy(
                fbuf[b],
                fouts[b].at[pl.ds(base + ch * _FCHUNK, _FCHUNK)],
                fosem[b])

    def gather_unit(outT, c):
        wb = {}
        for k in range(_NOCHUNK):
            s = k % 2
            h = wb.pop(k - 2, None)
            if h is not None:
                h.wait()

            def groups(it, _):
                off = it * 128
                for g in range(8):
                    i = off + g * 16
                    iv = idxbuf[pl.ds(k * _OCHUNK + i, 16)]
                    vals = plsc.load_gather(stage, [iv])
                    ochunk[s][pl.ds(i, 16)] = vals
                return 0

            lax.fori_loop(0, _OCHUNK // 128, groups, 0)
            wb[k] = pltpu.async_copy(
                ochunk[s], outT.at[c, pl.ds(k * _OCHUNK, _OCHUNK)], osem[s])
        for k in (_NOCHUNK - 2, _NOCHUNK - 1):
            wb[k].wait()

    fire_stage(0)
    for j in range(_MAXJ):
        u = wid + _NW * j

        @pl.when(u < C)
        def _():
            wait_stage()
            gather_unit(wlT_out, u)

        @pl.when(jnp.logical_and(u >= C, u < _NUNIT))
        def _():
            wait_stage()
            gather_unit(slT_out, u - C)

        if j + 1 < _MAXJ:
            fire_stage(j + 1)
        for ch in range(_NFCHUNK * j // _MAXJ,
                        _NFCHUNK * (j + 1) // _MAXJ):
            service_feature(ch)
    for b in range(2):
        h = state["fw"].pop(b, None)
        if h is not None:
            h.wait()


def kernel(x, index, weak_logits_mem, weak_features_mem,
           strong_logits_mem, strong_features_mem):
    wlT, slT, wf, sf = _gather4(weak_logits_mem.T, strong_logits_mem.T,
                                weak_features_mem, strong_features_mem,
                                index)
    return ([wlT.T, slT.T], [wf, sf])


# speedup vs baseline: 1.2330x; 1.2330x over previous
import functools

import jax
import jax.numpy as jnp
from jax import lax
from jax.experimental import pallas as pl
from jax.experimental.pallas import tpu as pltpu
from jax.experimental.pallas import tpu_sc as plsc

M = 100000
C = 100
D = 128
B = 16384

_info = plsc.get_sparse_core_info()
_NC = _info.num_cores
_NS = _info.num_subcores
_NW = _NC * _NS
_BPW = B // _NW
_FCHUNK = 128
_NFCHUNK = _BPW // _FCHUNK
_NUNIT = 2 * C
_MAXJ = (_NUNIT + _NW - 1) // _NW
_OCHUNK = 2048
_NOCHUNK = B // _OCHUNK

_mesh = plsc.VectorSubcoreMesh(core_axis_name="c", subcore_axis_name="s")


@functools.partial(
    pl.kernel,
    mesh=_mesh,
    compiler_params=pltpu.CompilerParams(needs_layout_passes=False),
    out_type=[
        jax.ShapeDtypeStruct((C, B), jnp.float32),
        jax.ShapeDtypeStruct((C, B), jnp.float32),
        jax.ShapeDtypeStruct((B, D), jnp.float32),
        jax.ShapeDtypeStruct((B, D), jnp.float32),
    ],
    scratch_types=[],
)
def _gather4(wlT, slT, wf_hbm, sf_hbm, idx_hbm,
             wlT_out, slT_out, wf_out, sf_out):
    wid = lax.axis_index("s") * _NC + lax.axis_index("c")
    base = wid * _BPW

    def feat_phase(fidx, fbuf, gsem, osem):
        stages = [
            pltpu.async_copy(
                idx_hbm.at[pl.ds(base + ch * _FCHUNK, _FCHUNK)],
                fidx[ch], gsem[0])
            for ch in range(_NFCHUNK)
        ]
        for h in stages:
            h.wait()
        fbanks = (wf_hbm, sf_hbm)
        fouts = (wf_out, sf_out)
        wb = {}
        for ch in range(_NFCHUNK):
            for b in range(2):
                h = wb.pop((b, ch - 1), None)
                if h is not None:
                    h.wait()
            gs = [pltpu.async_copy(fbanks[b].at[fidx[ch]], fbuf[b], gsem[b])
                  for b in range(2)]
            for b in range(2):
                gs[b].wait()
                wb[(b, ch)] = pltpu.async_copy(
                    fbuf[b],
                    fouts[b].at[pl.ds(base + ch * _FCHUNK, _FCHUNK)],
                    osem[b])
        for b in range(2):
            wb[(b, _NFCHUNK - 1)].wait()

    def logit_phase(idxbuf, stage, ochunk, ssem, osem):
        pltpu.async_copy(idx_hbm, idxbuf, ssem).wait()

        def run_unit(bankT, outT, c):
            pltpu.async_copy(bankT.at[c], stage, ssem).wait()
            wb = {}
            for k in range(_NOCHUNK):
                s = k % 2
                h = wb.pop(k - 2, None)
                if h is not None:
                    h.wait()

                def groups(it, _):
                    off = it * 64
                    for g in range(4):
                        i = off + g * 16
                        iv = idxbuf[pl.ds(k * _OCHUNK + i, 16)]
                        vals = plsc.load_gather(stage, [iv])
                        ochunk[s][pl.ds(i, 16)] = vals
                    return 0

                lax.fori_loop(0, _OCHUNK // 64, groups, 0)
                wb[k] = pltpu.async_copy(
                    ochunk[s],
                    outT.at[c, pl.ds(k * _OCHUNK, _OCHUNK)],
                    osem[s])
            for k in (_NOCHUNK - 2, _NOCHUNK - 1):
                wb[k].wait()

        for j in range(_MAXJ):
            u = wid + _NW * j

            @pl.when(u < C)
            def _():
                run_unit(wlT, wlT_out, u)

            @pl.when(jnp.logical_and(u >= C, u < 2 * C))
            def _():
                run_unit(slT, slT_out, u - C)

    pl.run_scoped(
        feat_phase,
        [pltpu.VMEM((_FCHUNK,), jnp.int32) for _ in range(_NFCHUNK)],
        [pltpu.VMEM((_FCHUNK, D), jnp.float32) for _ in range(2)],
        [pltpu.SemaphoreType.DMA for _ in range(2)],
        [pltpu.SemaphoreType.DMA for _ in range(2)],
    )
    pl.run_scoped(
        logit_phase,
        pltpu.VMEM((B,), jnp.int32),
        pltpu.VMEM((M,), jnp.float32),
        [pltpu.VMEM((_OCHUNK,), jnp.float32) for _ in range(2)],
        pltpu.SemaphoreType.DMA,
        [pltpu.SemaphoreType.DMA for _ in range(2)],
    )


def kernel(x, index, weak_logits_mem, weak_features_mem,
           strong_logits_mem, strong_features_mem):
    wlT, slT, wf, sf = _gather4(weak_logits_mem.T, strong_logits_mem.T,
                                weak_features_mem, strong_features_mem,
                                index)
    return ([wlT.T, slT.T], [wf, sf])
